# baseline (speedup 1.0000x reference)
import functools

import jax
import jax.numpy as jnp
from jax import lax
from jax.experimental import pallas as pl
from jax.experimental.pallas import tpu as pltpu
from jax.experimental.pallas import tpu_sc as plsc

NUM_STRANDS = 4
HIDDEN = 128
LAYERS = 2
B = 1024
L = 20
NW = 32
BPW = B // NW
NG = BPW // 16


def _sc_compose(braids_hbm, lut_hbm, out_hbm, braids_v, lut_v, out_v):
    wid = lax.axis_index("s") * 2 + lax.axis_index("c")
    pltpu.sync_copy(braids_hbm.at[wid], braids_v)
    pltpu.sync_copy(lut_hbm, lut_v)
    zero_i = jnp.zeros((16,), jnp.int32)
    one_i = jnp.full((16,), 1, jnp.int32)
    three_i = jnp.full((16,), 3, jnp.int32)
    zero_f = jnp.zeros((16,), jnp.float32)
    one_f = jnp.full((16,), 1.0, jnp.float32)
    neg_one_f = jnp.full((16,), -1.0, jnp.float32)
    for layer in range(LAYERS):
        cv = [lut_v[layer * 6 + k] for k in range(3)]
        sv = [lut_v[layer * 6 + 3 + k] for k in range(3)]
        for g in range(NG):
            ident = [one_f if e % 5 == 0 else zero_f for e in range(16)]

            def step(t, m, g=g, cv=cv, sv=sv):
                gen = braids_v[t * NG + g]
                mask = gen != zero_i
                p = jnp.maximum(jnp.abs(gen) - one_i, zero_i)
                sgn = jnp.where(gen < zero_i, neg_one_f, one_f)
                cth = jnp.where(p == zero_i, cv[0],
                                jnp.where(p == one_i, cv[1], cv[2]))
                sth = jnp.where(p == zero_i, sv[0],
                                jnp.where(p == one_i, sv[1], sv[2])) * sgn
                m = list(m)
                for pp in range(3):
                    pp_i = jnp.full((16,), pp, jnp.int32)
                    sel = mask & (p == pp_i)
                    c = jnp.where(sel, cth, one_f)
                    s = jnp.where(sel, sth, zero_f)
                    for j in range(4):
                        u = m[pp * 4 + j]
                        v = m[(pp + 1) * 4 + j]
                        m[pp * 4 + j] = u * c - v * s
                        m[(pp + 1) * 4 + j] = u * s + v * c
                return tuple(m)

            m = lax.fori_loop(0, L, step, tuple(ident))
            for e in range(16):
                out_v[layer * 16 + e, pl.ds(g * 16, 16)] = m[e]
    pltpu.sync_copy(out_v, out_hbm.at[wid])


def _layernorm(x, gamma, beta):
    mean = jnp.mean(x, axis=1, keepdims=True)
    cen = x - mean
    var = jnp.mean(cen * cen, axis=1, keepdims=True)
    return cen * jax.lax.rsqrt(var + 1e-5) * gamma + beta


def _dense_kernel(m0_ref, m1_ref, init_ref, g_ref, b_ref,
                  w1_ref, b1_ref, w2_ref, b2_ref, w3_ref, b3_ref, out_ref):
    s1 = []
    for i in range(4):
        acc = m0_ref[:, 4 * i:4 * i + 1] * init_ref[0:1, :]
        for j in range(1, 4):
            acc = acc + m0_ref[:, 4 * i + j:4 * i + j + 1] * init_ref[j:j + 1, :]
        s1.append(acc)
    g0 = g_ref[0:1, :]
    b0 = b_ref[0:1, :]
    s1 = [_layernorm(x, g0, b0) for x in s1]
    s2 = []
    for i in range(4):
        acc = m1_ref[:, 4 * i:4 * i + 1] * s1[0]
        for j in range(1, 4):
            acc = acc + m1_ref[:, 4 * i + j:4 * i + j + 1] * s1[j]
        s2.append(acc)
    g1 = g_ref[1:2, :]
    b1n = b_ref[1:2, :]
    s2 = [_layernorm(x, g1, b1n) for x in s2]
    h1 = b1_ref[0:1, :]
    for j in range(4):
        h1 = h1 + jnp.dot(s2[j], w1_ref[128 * j:128 * (j + 1), :],
                          preferred_element_type=jnp.float32)
    h1 = jnp.maximum(h1, 0.0)
    h2 = jnp.dot(h1, w2_ref[...], preferred_element_type=jnp.float32)
    h2 = jnp.maximum(h2 + b2_ref[0:1, :], 0.0)
    z0 = jnp.sum(h2 * w3_ref[0:1, :], axis=1, keepdims=True) + b3_ref[0]
    z1 = jnp.sum(h2 * w3_ref[1:2, :], axis=1, keepdims=True) + b3_ref[1]
    out_ref[:, 0:1] = jax.nn.sigmoid(z0)
    out_ref[:, 1:2] = z1


def kernel(braids, initial_state, thetas, ln_gamma, ln_beta,
           w1, b1, w2, b2, w3, b3):
    braids_w = braids.reshape(NW, NG, 16, L).transpose(0, 3, 1, 2).reshape(
        NW, L * NG, 16)
    lut = jnp.broadcast_to(
        jnp.concatenate([jnp.cos(thetas), jnp.sin(thetas)], axis=1).reshape(
            LAYERS * 6, 1), (LAYERS * 6, 16))
    sc_fn = functools.partial(
        pl.kernel,
        mesh=plsc.VectorSubcoreMesh(core_axis_name="c", subcore_axis_name="s"),
        out_type=jax.ShapeDtypeStruct((NW, 2 * 16, BPW), jnp.float32),
        scratch_types=[
            pltpu.VMEM((L * NG, 16), jnp.int32),
            pltpu.VMEM((LAYERS * 6, 16), jnp.float32),
            pltpu.VMEM((2 * 16, BPW), jnp.float32),
        ],
    )(_sc_compose)
    ms = sc_fn(braids_w, lut)
    ms = ms.reshape(NW, 2, 16, BPW).transpose(1, 0, 3, 2).reshape(2, B, 16)
    out = pl.pallas_call(
        _dense_kernel,
        out_shape=jax.ShapeDtypeStruct((B, 2), jnp.float32),
        in_specs=[pl.BlockSpec(memory_space=pltpu.VMEM)] * 10
        + [pl.BlockSpec(memory_space=pltpu.SMEM)],
        out_specs=pl.BlockSpec(memory_space=pltpu.VMEM),
    )(ms[0], ms[1], initial_state, ln_gamma, ln_beta,
      w1.T, b1.reshape(1, 128), w2.T, b2.reshape(1, 64), w3, b3)
    return out[:, 0], out[:, 1]
